# Initial kernel scaffold; baseline (speedup 1.0000x reference)
import jax
import jax.numpy as jnp
from jax.experimental import pallas as pl


def kernel(x, adj, weight, a, bias):
    raise NotImplementedError("write your pallas kernel here")


# speedup vs baseline: 1.3460x; 1.3460x over previous
import jax
import jax.numpy as jnp
from jax.experimental import pallas as pl
from jax.experimental.pallas import tpu as pltpu

M = 65
N = 100000
F = 128
NUM_HEAD = 8
SLOPE = 0.2
BLK = 4096
NBLK = (N + BLK - 1) // BLK
INT_MAX = jnp.iinfo(jnp.int32).max


def _argmax_body(adj_ref, idx_out_ref, max_sc, idx_sc):
    j = pl.program_id(0)

    @pl.when(j == 0)
    def _init():
        max_sc[...] = jnp.full((M, 1), -1.0, jnp.float32)
        idx_sc[...] = jnp.zeros((M, 1), jnp.int32)

    block = adj_ref[...]
    cols = j * BLK + jax.lax.broadcasted_iota(jnp.int32, (M, BLK), 1)
    vals = jnp.where(cols < N, block, -1.0)
    bmax = jnp.max(vals, axis=1, keepdims=True)
    cand = jnp.where(vals == bmax, cols, INT_MAX)
    bidx = jnp.min(cand, axis=1, keepdims=True)
    better = bmax > max_sc[...]
    max_sc[...] = jnp.where(better, bmax, max_sc[...])
    idx_sc[...] = jnp.where(better, bidx, idx_sc[...])

    @pl.when(j == NBLK - 1)
    def _fin():
        idx_out_ref[...] = idx_sc[...]


def _gat_body(idx_ref, x_ref, w_ref, a_ref, b_ref, out_ref, rows_sc, sem):
    def issue(i, _):
        pltpu.make_async_copy(
            x_ref.at[pl.ds(idx_ref[i], 1), :],
            rows_sc.at[pl.ds(i, 1), :], sem).start()
        return 0
    jax.lax.fori_loop(0, M, issue, 0)

    def drain(i, _):
        pltpu.make_async_copy(
            x_ref.at[pl.ds(idx_ref[i], 1), :],
            rows_sc.at[pl.ds(i, 1), :], sem).wait()
        return 0
    jax.lax.fori_loop(0, M, drain, 0)

    sel = rows_sc[...]
    h = jnp.dot(sel, w_ref[...], preferred_element_type=jnp.float32)
    a0 = a_ref[0:1, :]
    a1 = a_ref[1:2, :]
    c = jnp.sum(h[0:1, :] * a0)
    d = jnp.sum(h * a1, axis=1, keepdims=True)
    lg = c + d
    lg = jnp.where(lg >= 0, lg, SLOPE * lg)
    rows = jax.lax.broadcasted_iota(jnp.int32, (M, 1), 0)
    e = jnp.where(rows >= 1, jnp.exp(lg), 0.0)
    alpha = e / jnp.sum(e)
    hp = jnp.sum(alpha * h, axis=0, keepdims=True) + b_ref[...]
    out_ref[...] = jnp.broadcast_to(hp, (NUM_HEAD, F))


def kernel(x, adj, weight, a, bias):
    idx2 = pl.pallas_call(
        _argmax_body,
        grid=(NBLK,),
        in_specs=[pl.BlockSpec((M, BLK), lambda j: (0, j))],
        out_specs=pl.BlockSpec((M, 1), lambda j: (0, 0)),
        out_shape=jax.ShapeDtypeStruct((M, 1), jnp.int32),
        scratch_shapes=[pltpu.VMEM((M, 1), jnp.float32),
                        pltpu.VMEM((M, 1), jnp.int32)],
    )(adj)
    idx = idx2.reshape(M)

    out = pl.pallas_call(
        _gat_body,
        grid_spec=pltpu.PrefetchScalarGridSpec(
            num_scalar_prefetch=1,
            grid=(1,),
            in_specs=[
                pl.BlockSpec(memory_space=pl.ANY),
                pl.BlockSpec((F, F), lambda i, idx_ref: (0, 0)),
                pl.BlockSpec((2, F), lambda i, idx_ref: (0, 0)),
                pl.BlockSpec((1, F), lambda i, idx_ref: (0, 0)),
            ],
            out_specs=pl.BlockSpec((NUM_HEAD, F), lambda i, idx_ref: (0, 0)),
            scratch_shapes=[pltpu.VMEM((M, F), jnp.float32),
                            pltpu.SemaphoreType.DMA],
        ),
        out_shape=jax.ShapeDtypeStruct((NUM_HEAD, F), jnp.float32),
    )(idx, x, weight, a.reshape(2, F), bias.reshape(1, F))
    return out.reshape(NUM_HEAD * F)
